# Initial kernel scaffold; baseline (speedup 1.0000x reference)
import jax
import jax.numpy as jnp
from jax.experimental import pallas as pl


def kernel(predict, target):
    raise NotImplementedError("write your pallas kernel here")


# speedup vs baseline: 1.0160x; 1.0160x over previous
import functools

import jax
import jax.numpy as jnp
from jax.experimental import pallas as pl
from jax.experimental.pallas import tpu as pltpu

_BN = 512


def _chamfer_body(p_ref, t_ref, out_ref, colmin_ref, acc_ref):
    i = pl.program_id(0)
    nsteps = pl.num_programs(0)

    pblk = p_ref[...]
    tmat = t_ref[...]

    g = jnp.dot(pblk, tmat, preferred_element_type=jnp.float32,
                precision=jax.lax.Precision.HIGHEST)
    pn = jnp.sum(pblk * pblk, axis=1, keepdims=True)
    tn = jnp.sum(tmat * tmat, axis=0, keepdims=True)
    d = (pn - 2.0 * g) + tn

    dist1_sum = jnp.sum(jnp.min(d, axis=1))
    colmin = jnp.min(d, axis=0, keepdims=True)

    @pl.when(i == 0)
    def _init():
        acc_ref[0, 0] = dist1_sum
        colmin_ref[...] = colmin

    @pl.when(i > 0)
    def _update():
        acc_ref[0, 0] += dist1_sum
        colmin_ref[...] = jnp.minimum(colmin_ref[...], colmin)

    @pl.when(i == nsteps - 1)
    def _finish():
        n = nsteps * pblk.shape[0]
        m = tmat.shape[1]
        loss = acc_ref[0, 0] / n + jnp.sum(colmin_ref[...]) / m
        out_ref[...] = jnp.reshape(loss, (1, 1))


@functools.partial(jax.jit, static_argnames=())
def kernel(predict, target):
    p = predict[0]
    t = target[0]
    n, _ = p.shape
    m, _ = t.shape

    p8 = jnp.pad(p, ((0, 0), (0, 5)))
    t8 = jnp.pad(t, ((0, 0), (0, 5))).T

    out = pl.pallas_call(
        _chamfer_body,
        grid=(n // _BN,),
        in_specs=[
            pl.BlockSpec((_BN, 8), lambda i: (i, 0)),
            pl.BlockSpec((8, m), lambda i: (0, 0)),
        ],
        out_specs=pl.BlockSpec((1, 1), lambda i: (0, 0)),
        out_shape=jax.ShapeDtypeStruct((1, 1), jnp.float32),
        scratch_shapes=[
            pltpu.VMEM((1, m), jnp.float32),
            pltpu.SMEM((1, 1), jnp.float32),
        ],
    )(p8, t8)
    return out[0, 0]


# speedup vs baseline: 3.2653x; 3.2139x over previous
import functools

import jax
import jax.numpy as jnp
from jax.experimental import pallas as pl
from jax.experimental.pallas import tpu as pltpu

_BN = 512


def _chamfer_body(a_ref, b_ref, out_ref, colmin_ref, acc_ref):
    i = pl.program_id(0)
    nsteps = pl.num_programs(0)

    ablk = a_ref[...]
    bmat = b_ref[...]

    g = jnp.dot(ablk, bmat, preferred_element_type=jnp.float32)

    a32 = ablk.astype(jnp.float32)
    b32 = bmat.astype(jnp.float32)
    p_rec = a32[:, 0:3] + a32[:, 6:9]
    t_rec = b32[0:3, :] + b32[3:6, :]
    pn = jnp.sum(p_rec * p_rec, axis=1, keepdims=True)
    tn = jnp.sum(t_rec * t_rec, axis=0, keepdims=True)
    d = (pn - 2.0 * g) + tn

    dist1_sum = jnp.sum(jnp.min(d, axis=1))
    colmin = jnp.min(d, axis=0, keepdims=True)

    @pl.when(i == 0)
    def _init():
        acc_ref[0, 0] = dist1_sum
        colmin_ref[...] = colmin

    @pl.when(i > 0)
    def _update():
        acc_ref[0, 0] += dist1_sum
        colmin_ref[...] = jnp.minimum(colmin_ref[...], colmin)

    @pl.when(i == nsteps - 1)
    def _finish():
        n = nsteps * ablk.shape[0]
        m = bmat.shape[1]
        loss = acc_ref[0, 0] / n + jnp.sum(colmin_ref[...]) / m
        out_ref[...] = jnp.reshape(loss, (1, 1))


def _split_hi_lo(x):
    hi = x.astype(jnp.bfloat16)
    lo = (x - hi.astype(jnp.float32)).astype(jnp.bfloat16)
    return hi, lo


@functools.partial(jax.jit, static_argnames=())
def kernel(predict, target):
    p = predict[0]
    t = target[0]
    n, _ = p.shape
    m, _ = t.shape

    p_hi, p_lo = _split_hi_lo(p)
    t_hi, t_lo = _split_hi_lo(t)
    zcols = jnp.zeros((n, 4), jnp.bfloat16)
    a = jnp.concatenate([p_hi, p_hi, p_lo, p_lo, zcols], axis=1)
    b = jnp.concatenate([t_hi, t_lo, t_hi, t_lo,
                         jnp.zeros((m, 4), jnp.bfloat16)], axis=1).T

    out = pl.pallas_call(
        _chamfer_body,
        grid=(n // _BN,),
        in_specs=[
            pl.BlockSpec((_BN, 16), lambda i: (i, 0)),
            pl.BlockSpec((16, m), lambda i: (0, 0)),
        ],
        out_specs=pl.BlockSpec((1, 1), lambda i: (0, 0)),
        out_shape=jax.ShapeDtypeStruct((1, 1), jnp.float32),
        scratch_shapes=[
            pltpu.VMEM((1, m), jnp.float32),
            pltpu.SMEM((1, 1), jnp.float32),
        ],
    )(a, b)
    return out[0, 0]


# speedup vs baseline: 4.2698x; 1.3076x over previous
import functools

import jax
import jax.numpy as jnp
from jax.experimental import pallas as pl
from jax.experimental.pallas import tpu as pltpu

_BN = 512


def _chamfer_body(a_ref, b_ref, out_ref, colmax_ref, acc_ref):
    i = pl.program_id(0)
    nsteps = pl.num_programs(0)

    ablk = a_ref[...]
    bmat = b_ref[...]

    g = jnp.dot(ablk, bmat, preferred_element_type=jnp.float32)

    rowmax_sum = jnp.sum(jnp.max(g, axis=1))
    colmax = jnp.max(g, axis=0, keepdims=True)

    @pl.when(i == 0)
    def _init():
        acc_ref[0, 0] = rowmax_sum
        colmax_ref[...] = colmax

    @pl.when(i > 0)
    def _update():
        acc_ref[0, 0] += rowmax_sum
        colmax_ref[...] = jnp.maximum(colmax_ref[...], colmax)

    @pl.when(i == nsteps - 1)
    def _finish():
        n = nsteps * ablk.shape[0]
        m = bmat.shape[1]
        loss = -2.0 * (acc_ref[0, 0] / n + jnp.sum(colmax_ref[...]) / m)
        out_ref[...] = jnp.reshape(loss, (1, 1))


def _split_hi_lo(x):
    hi = x.astype(jnp.bfloat16)
    lo = (x - hi.astype(jnp.float32)).astype(jnp.bfloat16)
    return hi, lo


@functools.partial(jax.jit, static_argnames=())
def kernel(predict, target):
    p = predict[0]
    t = target[0]
    n, _ = p.shape
    m, _ = t.shape

    p_hi, p_lo = _split_hi_lo(p)
    t_hi, t_lo = _split_hi_lo(t)
    p_rec = p_hi.astype(jnp.float32) + p_lo.astype(jnp.float32)
    t_rec = t_hi.astype(jnp.float32) + t_lo.astype(jnp.float32)
    pn = 0.5 * jnp.sum(p_rec * p_rec, axis=1, keepdims=True)
    tn = 0.5 * jnp.sum(t_rec * t_rec, axis=1, keepdims=True)
    pn_hi, pn_lo = _split_hi_lo(pn)
    tn_hi, tn_lo = _split_hi_lo(tn)

    one_n = jnp.ones((n, 1), jnp.bfloat16)
    one_m = jnp.ones((m, 1), jnp.bfloat16)
    a = jnp.concatenate(
        [p_hi, p_hi, p_lo, p_lo, pn_hi, pn_lo, one_n, one_n], axis=1)
    b = jnp.concatenate(
        [t_hi, t_lo, t_hi, t_lo, -one_m, -one_m, -tn_hi, -tn_lo], axis=1).T

    out = pl.pallas_call(
        _chamfer_body,
        grid=(n // _BN,),
        in_specs=[
            pl.BlockSpec((_BN, 16), lambda i: (i, 0)),
            pl.BlockSpec((16, m), lambda i: (0, 0)),
        ],
        out_specs=pl.BlockSpec((1, 1), lambda i: (0, 0)),
        out_shape=jax.ShapeDtypeStruct((1, 1), jnp.float32),
        scratch_shapes=[
            pltpu.VMEM((1, m), jnp.float32),
            pltpu.SMEM((1, 1), jnp.float32),
        ],
    )(a, b)
    return out[0, 0]
